# baseline (speedup 1.0000x reference)
import functools

import jax
import jax.numpy as jnp
from jax import lax
from jax.experimental import pallas as pl
from jax.experimental.pallas import tpu as pltpu
from jax.experimental.pallas import tpu_sc as plsc

E = 160000
N = 10000
HIDDEN = 256


BLK_E = 640


def _silu(x):
    return x * (1.0 / (1.0 + jnp.exp(-x)))


def _dot(h, w_ref):
    return jax.lax.dot_general(h.astype(jnp.bfloat16), w_ref[...],
                               (((1,), (0,)), ((), ())),
                               preferred_element_type=jnp.float32)


def _mlp_body(ff_ref, ev_ref, w0_ref, w1_ref, w2_ref, w3_ref, w4_ref,
              b_ref, out_ref):
    h = ff_ref[...]
    h = _silu(_dot(h, w0_ref) + b_ref[0, :])
    h = _silu(_dot(h, w1_ref) + b_ref[1, :])
    h = _silu(_dot(h, w2_ref) + b_ref[2, :])
    h = _silu(_dot(h, w3_ref) + b_ref[3, :])
    scale = jnp.sum(h * w4_ref[...], axis=1, keepdims=True) + b_ref[4, 0]
    out_ref[...] = scale * ev_ref[...]


def _mlp_stage(ff, ev4, W0, W1, W2, W3, w4row, ballb4):
    grid = (E // BLK_E,)
    return pl.pallas_call(
        _mlp_body,
        grid=grid,
        in_specs=[
            pl.BlockSpec((BLK_E, HIDDEN), lambda i: (i, 0)),
            pl.BlockSpec((BLK_E, 8), lambda i: (i, 0)),
            pl.BlockSpec((HIDDEN, HIDDEN), lambda i: (0, 0)),
            pl.BlockSpec((HIDDEN, HIDDEN), lambda i: (0, 0)),
            pl.BlockSpec((HIDDEN, HIDDEN), lambda i: (0, 0)),
            pl.BlockSpec((HIDDEN, HIDDEN), lambda i: (0, 0)),
            pl.BlockSpec((1, HIDDEN), lambda i: (0, 0)),
            pl.BlockSpec((8, HIDDEN), lambda i: (0, 0)),
        ],
        out_specs=pl.BlockSpec((BLK_E, 8), lambda i: (i, 0)),
        out_shape=jax.ShapeDtypeStruct((E, 8), jnp.float32),
    )(ff, ev4, W0, W1, W2, W3, w4row, ballb4)


NC, NS = 2, 16
NW = NC * NS
EPW = E // NW
SCHUNK = 100
NSUB = EPW // SCHUNK
ZROWS = 640


def _scatter_body(val_hbm, idx_hbm, zero_hbm, out_hbm, idx_v, val_v, acc_sh):
    cid = lax.axis_index("c")
    sid = lax.axis_index("s")
    wid = sid * NC + cid

    zbase = sid * ZROWS
    last = N - (NS - 1) * ZROWS

    @pl.when(sid < NS - 1)
    def _():
        pltpu.sync_copy(zero_hbm.at[pl.ds(zbase, ZROWS)],
                        acc_sh.at[pl.ds(zbase, ZROWS)])

    @pl.when(sid == NS - 1)
    def _():
        pltpu.sync_copy(zero_hbm.at[pl.ds((NS - 1) * ZROWS, last)],
                        acc_sh.at[pl.ds((NS - 1) * ZROWS, last)])

    pltpu.sync_copy(idx_hbm.at[wid], idx_v)
    pltpu.sync_copy(val_hbm.at[wid], val_v)

    plsc.subcore_barrier()

    def _scat(j, _):
        pltpu.sync_copy(val_v.at[pl.ds(j * SCHUNK, SCHUNK)],
                        acc_sh.at[idx_v.at[j]], add=True)
        return 0
    lax.fori_loop(0, NSUB, _scat, 0)

    plsc.subcore_barrier()

    @pl.when(sid < NS - 1)
    def _():
        pltpu.sync_copy(acc_sh.at[pl.ds(zbase, ZROWS)],
                        out_hbm.at[cid, pl.ds(zbase, ZROWS)])

    @pl.when(sid == NS - 1)
    def _():
        pltpu.sync_copy(acc_sh.at[pl.ds((NS - 1) * ZROWS, last)],
                        out_hbm.at[cid, pl.ds((NS - 1) * ZROWS, last)])


def _scatter_stage(forces_e8, dst):
    val = forces_e8.reshape(NW, EPW, 8)
    idx4 = dst.astype(jnp.int32).reshape(NW, NSUB, SCHUNK)
    zeros = jnp.zeros((N, 8), jnp.float32)
    mesh = plsc.VectorSubcoreMesh(core_axis_name="c", subcore_axis_name="s")
    scat = pl.kernel(
        _scatter_body,
        out_type=jax.ShapeDtypeStruct((NC, N, 8), jnp.float32),
        mesh=mesh,
        compiler_params=pltpu.CompilerParams(use_tc_tiling_on_sc=False),
        scratch_types=[
            pltpu.VMEM((NSUB, SCHUNK), jnp.int32),
            pltpu.VMEM((EPW, 8), jnp.float32),
            pltpu.VMEM_SHARED((N, 8), jnp.float32),
        ],
    )
    return scat(val, idx4, zeros)


def _reduce_body(p_ref, out_ref):
    out_ref[...] = p_ref[0] + p_ref[1]


def _reduce_stage(partials):
    return pl.pallas_call(
        _reduce_body,
        out_shape=jax.ShapeDtypeStruct((N, 8), jnp.float32),
    )(partials)


@jax.jit
def kernel(force_features, edge_vectors, edge_index_dst, pos,
           W0, b0, W1, b1, W2, b2, W3, b3, W4, b4):
    ev8 = jnp.pad(edge_vectors, ((0, 0), (0, 5)))
    w4row = W4.reshape(1, HIDDEN)
    ballb4 = jnp.concatenate(
        [jnp.stack([b0, b1, b2, b3]),
         jnp.broadcast_to(b4.reshape(1, 1), (1, HIDDEN)),
         jnp.zeros((3, HIDDEN), jnp.float32)], axis=0)
    Wb = [w.astype(jnp.bfloat16) for w in (W0, W1, W2, W3)]
    forces_e8 = _mlp_stage(force_features, ev8, Wb[0], Wb[1], Wb[2], Wb[3], w4row, ballb4)
    partials = _scatter_stage(forces_e8, edge_index_dst)
    forces4 = _reduce_stage(partials)
    return forces4[:, :3]


# speedup vs baseline: 1.0286x; 1.0286x over previous
import functools

import jax
import jax.numpy as jnp
from jax import lax
from jax.experimental import pallas as pl
from jax.experimental.pallas import tpu as pltpu
from jax.experimental.pallas import tpu_sc as plsc

E = 160000
N = 10000
HIDDEN = 256


BLK_E = 640


def _silu(x):
    return 0.5 * x * (1.0 + jnp.tanh(0.5 * x))


def _dot(h, w_ref):
    return jax.lax.dot_general(h.astype(jnp.bfloat16), w_ref[...],
                               (((1,), (0,)), ((), ())),
                               preferred_element_type=jnp.float32)


def _mlp_body(ff_ref, ev_ref, w0_ref, w1_ref, w2_ref, w3_ref, w4_ref,
              b_ref, out_ref):
    h = ff_ref[...]
    h = _silu(_dot(h, w0_ref) + b_ref[0, :])
    h = _silu(_dot(h, w1_ref) + b_ref[1, :])
    h = _silu(_dot(h, w2_ref) + b_ref[2, :])
    h = _silu(_dot(h, w3_ref) + b_ref[3, :])
    scale = jnp.sum(h * w4_ref[...], axis=1, keepdims=True) + b_ref[4, 0]
    out_ref[...] = scale * ev_ref[...]


def _mlp_stage(ff, ev4, W0, W1, W2, W3, w4row, ballb4):
    grid = (E // BLK_E,)
    return pl.pallas_call(
        _mlp_body,
        grid=grid,
        in_specs=[
            pl.BlockSpec((BLK_E, HIDDEN), lambda i: (i, 0)),
            pl.BlockSpec((BLK_E, 8), lambda i: (i, 0)),
            pl.BlockSpec((HIDDEN, HIDDEN), lambda i: (0, 0)),
            pl.BlockSpec((HIDDEN, HIDDEN), lambda i: (0, 0)),
            pl.BlockSpec((HIDDEN, HIDDEN), lambda i: (0, 0)),
            pl.BlockSpec((HIDDEN, HIDDEN), lambda i: (0, 0)),
            pl.BlockSpec((1, HIDDEN), lambda i: (0, 0)),
            pl.BlockSpec((8, HIDDEN), lambda i: (0, 0)),
        ],
        out_specs=pl.BlockSpec((BLK_E, 8), lambda i: (i, 0)),
        out_shape=jax.ShapeDtypeStruct((E, 8), jnp.float32),
    )(ff, ev4, W0, W1, W2, W3, w4row, ballb4)


NC, NS = 2, 16
NW = NC * NS
EPW = E // NW
SCHUNK = 100
NSUB = EPW // SCHUNK
ZROWS = 640


def _scatter_body(val_hbm, idx_hbm, zero_hbm, out_hbm, idx_v, val_v, acc_sh):
    cid = lax.axis_index("c")
    sid = lax.axis_index("s")
    wid = sid * NC + cid

    zbase = sid * ZROWS
    last = N - (NS - 1) * ZROWS

    @pl.when(sid < NS - 1)
    def _():
        pltpu.sync_copy(zero_hbm.at[pl.ds(zbase, ZROWS)],
                        acc_sh.at[pl.ds(zbase, ZROWS)])

    @pl.when(sid == NS - 1)
    def _():
        pltpu.sync_copy(zero_hbm.at[pl.ds((NS - 1) * ZROWS, last)],
                        acc_sh.at[pl.ds((NS - 1) * ZROWS, last)])

    pltpu.sync_copy(idx_hbm.at[wid], idx_v)
    pltpu.sync_copy(val_hbm.at[wid], val_v)

    plsc.subcore_barrier()

    def _scat(j, _):
        pltpu.sync_copy(val_v.at[pl.ds(j * SCHUNK, SCHUNK)],
                        acc_sh.at[idx_v.at[j]], add=True)
        return 0
    lax.fori_loop(0, NSUB, _scat, 0)

    plsc.subcore_barrier()

    @pl.when(sid < NS - 1)
    def _():
        pltpu.sync_copy(acc_sh.at[pl.ds(zbase, ZROWS)],
                        out_hbm.at[cid, pl.ds(zbase, ZROWS)])

    @pl.when(sid == NS - 1)
    def _():
        pltpu.sync_copy(acc_sh.at[pl.ds((NS - 1) * ZROWS, last)],
                        out_hbm.at[cid, pl.ds((NS - 1) * ZROWS, last)])


def _scatter_stage(forces_e8, dst):
    val = forces_e8.reshape(NW, EPW, 8)
    idx4 = dst.astype(jnp.int32).reshape(NW, NSUB, SCHUNK)
    zeros = jnp.zeros((N, 8), jnp.float32)
    mesh = plsc.VectorSubcoreMesh(core_axis_name="c", subcore_axis_name="s")
    scat = pl.kernel(
        _scatter_body,
        out_type=jax.ShapeDtypeStruct((NC, N, 8), jnp.float32),
        mesh=mesh,
        compiler_params=pltpu.CompilerParams(use_tc_tiling_on_sc=False),
        scratch_types=[
            pltpu.VMEM((NSUB, SCHUNK), jnp.int32),
            pltpu.VMEM((EPW, 8), jnp.float32),
            pltpu.VMEM_SHARED((N, 8), jnp.float32),
        ],
    )
    return scat(val, idx4, zeros)


def _reduce_body(p_ref, out_ref):
    out_ref[...] = p_ref[0] + p_ref[1]


def _reduce_stage(partials):
    return pl.pallas_call(
        _reduce_body,
        out_shape=jax.ShapeDtypeStruct((N, 8), jnp.float32),
    )(partials)


@jax.jit
def kernel(force_features, edge_vectors, edge_index_dst, pos,
           W0, b0, W1, b1, W2, b2, W3, b3, W4, b4):
    ev8 = jnp.pad(edge_vectors, ((0, 0), (0, 5)))
    w4row = W4.reshape(1, HIDDEN)
    ballb4 = jnp.concatenate(
        [jnp.stack([b0, b1, b2, b3]),
         jnp.broadcast_to(b4.reshape(1, 1), (1, HIDDEN)),
         jnp.zeros((3, HIDDEN), jnp.float32)], axis=0)
    Wb = [w.astype(jnp.bfloat16) for w in (W0, W1, W2, W3)]
    forces_e8 = _mlp_stage(force_features, ev8, Wb[0], Wb[1], Wb[2], Wb[3], w4row, ballb4)
    partials = _scatter_stage(forces_e8, edge_index_dst)
    forces4 = _reduce_stage(partials)
    return forces4[:, :3]


# speedup vs baseline: 1.0359x; 1.0071x over previous
import functools

import jax
import jax.numpy as jnp
from jax import lax
from jax.experimental import pallas as pl
from jax.experimental.pallas import tpu as pltpu
from jax.experimental.pallas import tpu_sc as plsc

E = 160000
N = 10000
HIDDEN = 256


BLK_E = 640


def _silu(x):
    return 0.5 * x * (1.0 + jnp.tanh(0.5 * x))


def _dot(h, w_ref):
    return jax.lax.dot_general(h.astype(jnp.bfloat16), w_ref[...],
                               (((1,), (0,)), ((), ())),
                               preferred_element_type=jnp.float32)


def _mlp_body(ff_ref, ev_ref, w0_ref, w1_ref, w2_ref, w3_ref, w4_ref,
              b_ref, out_ref):
    h = ff_ref[...]
    h = _silu(_dot(h, w0_ref) + b_ref[0, :])
    h = _silu(_dot(h, w1_ref) + b_ref[1, :])
    h = _silu(_dot(h, w2_ref) + b_ref[2, :])
    h = _silu(_dot(h, w3_ref) + b_ref[3, :])
    scale = jnp.sum(h * w4_ref[...], axis=1, keepdims=True) + b_ref[4, 0]
    out_ref[...] = scale * ev_ref[...]


def _mlp_stage(ff, ev4, W0, W1, W2, W3, w4row, ballb4):
    grid = (E // BLK_E,)
    return pl.pallas_call(
        _mlp_body,
        grid=grid,
        in_specs=[
            pl.BlockSpec((BLK_E, HIDDEN), lambda i: (i, 0)),
            pl.BlockSpec((BLK_E, 8), lambda i: (i, 0)),
            pl.BlockSpec((HIDDEN, HIDDEN), lambda i: (0, 0)),
            pl.BlockSpec((HIDDEN, HIDDEN), lambda i: (0, 0)),
            pl.BlockSpec((HIDDEN, HIDDEN), lambda i: (0, 0)),
            pl.BlockSpec((HIDDEN, HIDDEN), lambda i: (0, 0)),
            pl.BlockSpec((1, HIDDEN), lambda i: (0, 0)),
            pl.BlockSpec((8, HIDDEN), lambda i: (0, 0)),
        ],
        out_specs=pl.BlockSpec((BLK_E, 8), lambda i: (i, 0)),
        out_shape=jax.ShapeDtypeStruct((E, 8), jnp.float32),
    )(ff, ev4, W0, W1, W2, W3, w4row, ballb4)


NC, NS = 2, 16
NW = NC * NS
EPW = E // NW
SCHUNK = 100
NSUB = EPW // SCHUNK
ZROWS = 640


def _scatter_body(val_hbm, idx_hbm, zero_hbm, out_hbm, idx_v, val_v, acc_sh, sem):
    cid = lax.axis_index("c")
    sid = lax.axis_index("s")
    wid = sid * NC + cid

    zbase = sid * ZROWS
    last = N - (NS - 1) * ZROWS

    @pl.when(sid < NS - 1)
    def _():
        pltpu.sync_copy(zero_hbm.at[pl.ds(zbase, ZROWS)],
                        acc_sh.at[pl.ds(zbase, ZROWS)])

    @pl.when(sid == NS - 1)
    def _():
        pltpu.sync_copy(zero_hbm.at[pl.ds((NS - 1) * ZROWS, last)],
                        acc_sh.at[pl.ds((NS - 1) * ZROWS, last)])

    pltpu.sync_copy(idx_hbm.at[wid], idx_v)
    pltpu.sync_copy(val_hbm.at[wid], val_v)

    plsc.subcore_barrier()

    def _scat(j, _):
        pltpu.async_copy(val_v.at[pl.ds(j * SCHUNK, SCHUNK)],
                         acc_sh.at[idx_v.at[j]], sem, add=True)
        return 0
    lax.fori_loop(0, NSUB, _scat, 0)

    def _drain(j, _):
        pltpu.make_async_copy(val_v.at[pl.ds(j * SCHUNK, SCHUNK)],
                              acc_sh.at[idx_v.at[j]], sem).wait()
        return 0
    lax.fori_loop(0, NSUB, _drain, 0)

    plsc.subcore_barrier()

    @pl.when(sid < NS - 1)
    def _():
        pltpu.sync_copy(acc_sh.at[pl.ds(zbase, ZROWS)],
                        out_hbm.at[cid, pl.ds(zbase, ZROWS)])

    @pl.when(sid == NS - 1)
    def _():
        pltpu.sync_copy(acc_sh.at[pl.ds((NS - 1) * ZROWS, last)],
                        out_hbm.at[cid, pl.ds((NS - 1) * ZROWS, last)])


def _scatter_stage(forces_e8, dst):
    val = forces_e8.reshape(NW, EPW, 8)
    idx4 = dst.astype(jnp.int32).reshape(NW, NSUB, SCHUNK)
    zeros = jnp.zeros((N, 8), jnp.float32)
    mesh = plsc.VectorSubcoreMesh(core_axis_name="c", subcore_axis_name="s")
    scat = pl.kernel(
        _scatter_body,
        out_type=jax.ShapeDtypeStruct((NC, N, 8), jnp.float32),
        mesh=mesh,
        compiler_params=pltpu.CompilerParams(use_tc_tiling_on_sc=False),
        scratch_types=[
            pltpu.VMEM((NSUB, SCHUNK), jnp.int32),
            pltpu.VMEM((EPW, 8), jnp.float32),
            pltpu.VMEM_SHARED((N, 8), jnp.float32),
            pltpu.SemaphoreType.DMA,
        ],
    )
    return scat(val, idx4, zeros)


def _reduce_body(p_ref, out_ref):
    out_ref[...] = p_ref[0] + p_ref[1]


def _reduce_stage(partials):
    return pl.pallas_call(
        _reduce_body,
        out_shape=jax.ShapeDtypeStruct((N, 8), jnp.float32),
    )(partials)


@jax.jit
def kernel(force_features, edge_vectors, edge_index_dst, pos,
           W0, b0, W1, b1, W2, b2, W3, b3, W4, b4):
    ev8 = jnp.pad(edge_vectors, ((0, 0), (0, 5)))
    w4row = W4.reshape(1, HIDDEN)
    ballb4 = jnp.concatenate(
        [jnp.stack([b0, b1, b2, b3]),
         jnp.broadcast_to(b4.reshape(1, 1), (1, HIDDEN)),
         jnp.zeros((3, HIDDEN), jnp.float32)], axis=0)
    Wb = [w.astype(jnp.bfloat16) for w in (W0, W1, W2, W3)]
    forces_e8 = _mlp_stage(force_features, ev8, Wb[0], Wb[1], Wb[2], Wb[3], w4row, ballb4)
    partials = _scatter_stage(forces_e8, edge_index_dst)
    forces4 = _reduce_stage(partials)
    return forces4[:, :3]


# speedup vs baseline: 1.3511x; 1.3043x over previous
import jax
import jax.numpy as jnp
from jax import lax
from jax.experimental import pallas as pl
from jax.experimental.pallas import tpu as pltpu
from jax.experimental.pallas import tpu_sc as plsc

E = 160000
N = 10000
HIDDEN = 256


BLK_E = 640


def _silu(x):
    return 0.5 * x * (1.0 + jnp.tanh(0.5 * x))


def _dot(h, w_ref):
    return jax.lax.dot_general(h.astype(jnp.bfloat16), w_ref[...],
                               (((1,), (0,)), ((), ())),
                               preferred_element_type=jnp.float32)


def _mlp_body(ff_ref, w0_ref, w1_ref, w2_ref, w3_ref, w4_ref, b_ref, out_ref):
    h = ff_ref[...]
    h = _silu(_dot(h, w0_ref) + b_ref[0, :])
    h = _silu(_dot(h, w1_ref) + b_ref[1, :])
    h = _silu(_dot(h, w2_ref) + b_ref[2, :])
    h = _silu(_dot(h, w3_ref) + b_ref[3, :])
    out_ref[...] = jnp.sum(h * w4_ref[...], axis=1, keepdims=True) + b_ref[4, 0]


def _mlp_stage(ff, W0, W1, W2, W3, w4row, ballb4):
    grid = (E // BLK_E,)
    return pl.pallas_call(
        _mlp_body,
        grid=grid,
        in_specs=[
            pl.BlockSpec((BLK_E, HIDDEN), lambda i: (i, 0)),
            pl.BlockSpec((HIDDEN, HIDDEN), lambda i: (0, 0)),
            pl.BlockSpec((HIDDEN, HIDDEN), lambda i: (0, 0)),
            pl.BlockSpec((HIDDEN, HIDDEN), lambda i: (0, 0)),
            pl.BlockSpec((HIDDEN, HIDDEN), lambda i: (0, 0)),
            pl.BlockSpec((1, HIDDEN), lambda i: (0, 0)),
            pl.BlockSpec((8, HIDDEN), lambda i: (0, 0)),
        ],
        out_specs=pl.BlockSpec((BLK_E, 1), lambda i: (i, 0)),
        out_shape=jax.ShapeDtypeStruct((E, 1), jnp.float32),
    )(ff, W0, W1, W2, W3, w4row, ballb4)


NC, NS = 2, 16
NW = NC * NS
EPW = E // NW
EPW_PAD = EPW + 8
NGRP = (EPW + 15) // 16
SCHUNK = 100
NSUB = EPW // SCHUNK
ZROWS = 640


def _scatter_body(s_hbm, x_hbm, y_hbm, z_hbm, idx_hbm, zero_hbm, out_hbm,
                  s_v, x_v, y_v, z_v, idx_v, val_v, acc_sh, sem):
    cid = lax.axis_index("c")
    sid = lax.axis_index("s")
    wid = sid * NC + cid
    ebase = wid * EPW

    zbase = sid * ZROWS
    last = N - (NS - 1) * ZROWS

    @pl.when(sid < NS - 1)
    def _():
        pltpu.sync_copy(zero_hbm.at[pl.ds(zbase, ZROWS)],
                        acc_sh.at[pl.ds(zbase, ZROWS)])

    @pl.when(sid == NS - 1)
    def _():
        pltpu.sync_copy(zero_hbm.at[pl.ds((NS - 1) * ZROWS, last)],
                        acc_sh.at[pl.ds((NS - 1) * ZROWS, last)])

    pltpu.async_copy(s_hbm.at[pl.ds(ebase, EPW)], s_v.at[pl.ds(0, EPW)], sem)
    pltpu.async_copy(x_hbm.at[pl.ds(ebase, EPW)], x_v.at[pl.ds(0, EPW)], sem)
    pltpu.async_copy(y_hbm.at[pl.ds(ebase, EPW)], y_v.at[pl.ds(0, EPW)], sem)
    pltpu.async_copy(z_hbm.at[pl.ds(ebase, EPW)], z_v.at[pl.ds(0, EPW)], sem)
    pltpu.async_copy(idx_hbm.at[wid], idx_v, sem)
    pltpu.make_async_copy(s_hbm.at[pl.ds(ebase, EPW)], s_v.at[pl.ds(0, EPW)], sem).wait()
    pltpu.make_async_copy(x_hbm.at[pl.ds(ebase, EPW)], x_v.at[pl.ds(0, EPW)], sem).wait()
    pltpu.make_async_copy(y_hbm.at[pl.ds(ebase, EPW)], y_v.at[pl.ds(0, EPW)], sem).wait()
    pltpu.make_async_copy(z_hbm.at[pl.ds(ebase, EPW)], z_v.at[pl.ds(0, EPW)], sem).wait()
    pltpu.make_async_copy(idx_hbm.at[wid], idx_v, sem).wait()

    c0 = jnp.zeros((16,), jnp.int32)
    def _pack(g, _):
        base = g * 16
        rows = base + lax.iota(jnp.int32, 16)
        s = s_v[pl.ds(base, 16)]
        plsc.store_scatter(val_v, [rows, c0], s * x_v[pl.ds(base, 16)])
        plsc.store_scatter(val_v, [rows, c0 + 1], s * y_v[pl.ds(base, 16)])
        plsc.store_scatter(val_v, [rows, c0 + 2], s * z_v[pl.ds(base, 16)])
        return 0
    lax.fori_loop(0, NGRP, _pack, 0)

    plsc.subcore_barrier()

    def _scat(j, _):
        pltpu.async_copy(val_v.at[pl.ds(j * SCHUNK, SCHUNK)],
                         acc_sh.at[idx_v.at[j]], sem, add=True)
        return 0
    lax.fori_loop(0, NSUB, _scat, 0)

    def _drain(j, _):
        pltpu.make_async_copy(val_v.at[pl.ds(j * SCHUNK, SCHUNK)],
                              acc_sh.at[idx_v.at[j]], sem).wait()
        return 0
    lax.fori_loop(0, NSUB, _drain, 0)

    plsc.subcore_barrier()

    @pl.when(sid < NS - 1)
    def _():
        pltpu.sync_copy(acc_sh.at[pl.ds(zbase, ZROWS)],
                        out_hbm.at[cid, pl.ds(zbase, ZROWS)])

    @pl.when(sid == NS - 1)
    def _():
        pltpu.sync_copy(acc_sh.at[pl.ds((NS - 1) * ZROWS, last)],
                        out_hbm.at[cid, pl.ds((NS - 1) * ZROWS, last)])


def _scatter_stage(scale, evx, evy, evz, dst):
    idx3 = dst.astype(jnp.int32).reshape(NW, NSUB, SCHUNK)
    zeros = jnp.zeros((N, 8), jnp.float32)
    mesh = plsc.VectorSubcoreMesh(core_axis_name="c", subcore_axis_name="s")
    scat = pl.kernel(
        _scatter_body,
        out_type=jax.ShapeDtypeStruct((NC, N, 8), jnp.float32),
        mesh=mesh,
        compiler_params=pltpu.CompilerParams(use_tc_tiling_on_sc=False,
                                             needs_layout_passes=False),
        scratch_types=[
            pltpu.VMEM((EPW_PAD,), jnp.float32),
            pltpu.VMEM((EPW_PAD,), jnp.float32),
            pltpu.VMEM((EPW_PAD,), jnp.float32),
            pltpu.VMEM((EPW_PAD,), jnp.float32),
            pltpu.VMEM((NSUB, SCHUNK), jnp.int32),
            pltpu.VMEM((EPW_PAD, 8), jnp.float32),
            pltpu.VMEM_SHARED((N, 8), jnp.float32),
            pltpu.SemaphoreType.DMA,
        ],
    )
    return scat(scale, evx, evy, evz, idx3, zeros)


def _reduce_body(p_ref, out_ref):
    out_ref[...] = p_ref[0] + p_ref[1]


def _reduce_stage(partials):
    return pl.pallas_call(
        _reduce_body,
        out_shape=jax.ShapeDtypeStruct((N, 8), jnp.float32),
    )(partials)


@jax.jit
def kernel(force_features, edge_vectors, edge_index_dst, pos,
           W0, b0, W1, b1, W2, b2, W3, b3, W4, b4):
    w4row = W4.reshape(1, HIDDEN)
    ballb4 = jnp.concatenate(
        [jnp.stack([b0, b1, b2, b3]),
         jnp.broadcast_to(b4.reshape(1, 1), (1, HIDDEN)),
         jnp.zeros((3, HIDDEN), jnp.float32)], axis=0)
    Wb = [w.astype(jnp.bfloat16) for w in (W0, W1, W2, W3)]
    scale = _mlp_stage(force_features, Wb[0], Wb[1], Wb[2], Wb[3],
                       w4row, ballb4).reshape(E)
    evx = edge_vectors[:, 0]
    evy = edge_vectors[:, 1]
    evz = edge_vectors[:, 2]
    partials = _scatter_stage(scale, evx, evy, evz, edge_index_dst)
    forces8 = _reduce_stage(partials)
    return forces8[:, :3]
